# baseline (speedup 1.0000x reference)
import functools

import jax
import jax.numpy as jnp
from jax import lax
from jax.experimental import pallas as pl
from jax.experimental.pallas import tpu as pltpu
from jax.experimental.pallas import tpu_sc as plsc

_NC = 2
_NS = 16
_NW = _NC * _NS
_C = 2
_NBUF = 3
_LA = 2
_LANE = 128
_P = 2


@functools.lru_cache(maxsize=None)
def _build(B, V, ROW):
    SL = ROW // _LANE
    b_per_w = B // _NW
    n_chunks = b_per_w // _C
    mesh = plsc.VectorSubcoreMesh(core_axis_name="c", subcore_axis_name="s")

    @functools.partial(
        pl.kernel,
        mesh=mesh,
        out_type=jax.ShapeDtypeStruct((B, SL, _LANE), jnp.float32),
        scratch_types=[
            pltpu.VMEM((n_chunks, _C), jnp.int32),
            pltpu.VMEM((_NBUF, _C, SL, _LANE), jnp.float32),
        ]
        + [pltpu.SemaphoreType.DMA] * (2 * _NBUF),
    )
    def gather_k(idx_hbm, table_hbm, out_hbm, idx_v, bufs, *sems):
        gsem = sems[:_NBUF]
        wsem = sems[_NBUF:]
        wid = lax.axis_index("s") * _NC + lax.axis_index("c")
        base = wid * b_per_w
        pltpu.sync_copy(idx_hbm.at[wid], idx_v)

        gcp = [None] * _NBUF
        wcp = [None] * _NBUF

        def start_gather(c):
            b = c % _NBUF
            if wcp[b] is not None:
                wcp[b].wait()
                wcp[b] = None
            gcp[b] = pltpu.async_copy(
                table_hbm.at[idx_v.at[c]], bufs.at[b], gsem[b]
            )

        for c in range(min(_LA, n_chunks)):
            start_gather(c)
        for g in range(n_chunks):
            b = g % _NBUF
            gcp[b].wait()
            wcp[b] = pltpu.async_copy(
                bufs.at[b],
                out_hbm.at[pl.ds(base + g * _C, _C)],
                wsem[b],
            )
            nxt = g + _LA
            if nxt < n_chunks:
                start_gather(nxt)
        for b in range(_NBUF):
            if wcp[b] is not None:
                wcp[b].wait()

    return gather_k


def _tc_gather_body(idx_ref, x_ref, o_ref):
    o_ref[...] = x_ref[...]


@functools.lru_cache(maxsize=None)
def _tc_gather_build(n_rows, maps_shape):
    V = maps_shape[0]
    tail = maps_shape[1:]
    zeros = (0,) * len(tail)
    return pl.pallas_call(
        _tc_gather_body,
        grid_spec=pltpu.PrefetchScalarGridSpec(
            num_scalar_prefetch=1,
            grid=(n_rows,),
            in_specs=[
                pl.BlockSpec((1, *tail), lambda i, idx: (idx[i], *zeros))
            ],
            out_specs=pl.BlockSpec((1, *tail), lambda i, idx: (i, *zeros)),
        ),
        out_shape=jax.ShapeDtypeStruct((n_rows, *tail), jnp.float32),
    )


def kernel(position_ids, maps):
    B = position_ids.size
    V = maps.shape[0]
    tail = maps.shape[1:]
    ROW = 1
    for s in tail:
        ROW *= s
    table = maps.reshape(V, ROW // _LANE, _LANE)
    flat_ids = position_ids.astype(jnp.int32).reshape(B)
    half = B // 2
    ids1 = lax.slice_in_dim(flat_ids, 0, half, axis=0)
    ids2 = lax.slice_in_dim(flat_ids, half, B, axis=0)
    idx1 = ids1.reshape(_NW, (half // _NW) // _C, _C)
    o1 = _build(half, V, ROW)(idx1, table).reshape(half, *tail)
    o2 = _tc_gather_build(half, maps.shape)(ids2, maps)
    out = jnp.concatenate([o1, o2], axis=0)
    return out.reshape(*position_ids.shape, *tail)


# speedup vs baseline: 2.4530x; 2.4530x over previous
import functools

import jax
import jax.numpy as jnp
from jax import lax
from jax.experimental import pallas as pl
from jax.experimental.pallas import tpu as pltpu
from jax.experimental.pallas import tpu_sc as plsc

_NC = 2
_NS = 16
_NW = _NC * _NS
_C = 2
_NBUF = 3
_LA = 2
_LANE = 128


@functools.lru_cache(maxsize=None)
def _build(B, V, ROW):
    SL = ROW // _LANE
    b_per_w = B // _NW
    n_chunks = b_per_w // _C
    mesh = plsc.VectorSubcoreMesh(core_axis_name="c", subcore_axis_name="s")

    @functools.partial(
        pl.kernel,
        mesh=mesh,
        out_type=jax.ShapeDtypeStruct((B, SL, _LANE), jnp.float32),
        scratch_types=[
            pltpu.VMEM((n_chunks, _C), jnp.int32),
            pltpu.VMEM((_NBUF, _C, SL, _LANE), jnp.float32),
        ]
        + [pltpu.SemaphoreType.DMA] * (2 * _NBUF),
    )
    def gather_k(idx_hbm, table_hbm, out_hbm, idx_v, bufs, *sems):
        gsem = sems[:_NBUF]
        wsem = sems[_NBUF:]
        wid = lax.axis_index("s") * _NC + lax.axis_index("c")
        base = wid * b_per_w
        pltpu.sync_copy(idx_hbm.at[wid], idx_v)

        gcp = [None] * _NBUF
        wcp = [None] * _NBUF

        def start_gather(c):
            b = c % _NBUF
            if wcp[b] is not None:
                wcp[b].wait()
                wcp[b] = None
            gcp[b] = pltpu.async_copy(
                table_hbm.at[idx_v.at[c]], bufs.at[b], gsem[b]
            )

        for c in range(min(_LA, n_chunks)):
            start_gather(c)
        for g in range(n_chunks):
            b = g % _NBUF
            gcp[b].wait()
            wcp[b] = pltpu.async_copy(
                bufs.at[b],
                out_hbm.at[pl.ds(base + g * _C, _C)],
                wsem[b],
            )
            nxt = g + _LA
            if nxt < n_chunks:
                start_gather(nxt)
        for b in range(_NBUF):
            if wcp[b] is not None:
                wcp[b].wait()

    return gather_k


def kernel(position_ids, maps):
    B = position_ids.size
    V = maps.shape[0]
    tail = maps.shape[1:]
    ROW = 1
    for s in tail:
        ROW *= s
    table = maps.reshape(V, ROW // _LANE, _LANE)
    flat_ids = position_ids.astype(jnp.int32).reshape(B)
    idx = flat_ids.reshape(_NW, (B // _NW) // _C, _C)
    out = _build(B, V, ROW)(idx, table)
    return out.reshape(*position_ids.shape, *tail)


# speedup vs baseline: 4.4603x; 1.8183x over previous
import functools

import jax
import jax.numpy as jnp
from jax import lax
from jax.experimental import pallas as pl
from jax.experimental.pallas import tpu as pltpu
from jax.experimental.pallas import tpu_sc as plsc

_NC = 2
_NS = 16
_NW = _NC * _NS
_C = 2
_NBUF = 3
_LA = 2
_LANE = 128


@functools.lru_cache(maxsize=None)
def _build(batch_shape, V, ROW):
    NB, SEQ = batch_shape
    SL = ROW // _LANE
    B = NB * SEQ
    b_per_w = B // _NW
    n_chunks = b_per_w // _C
    w_per_row = SEQ // b_per_w
    mesh = plsc.VectorSubcoreMesh(core_axis_name="c", subcore_axis_name="s")

    @functools.partial(
        pl.kernel,
        mesh=mesh,
        out_type=jax.ShapeDtypeStruct((NB, SEQ, SL, _LANE), jnp.float32),
        scratch_types=[
            pltpu.VMEM((n_chunks, _C), jnp.int32),
            pltpu.VMEM((_NBUF, _C, SL, _LANE), jnp.float32),
        ]
        + [pltpu.SemaphoreType.DMA] * (2 * _NBUF),
    )
    def gather_k(idx_hbm, table_hbm, out_hbm, idx_v, bufs, *sems):
        gsem = sems[:_NBUF]
        wsem = sems[_NBUF:]
        wid = lax.axis_index("s") * _NC + lax.axis_index("c")
        bi = wid // w_per_row
        base = (wid % w_per_row) * b_per_w
        pltpu.sync_copy(idx_hbm.at[wid], idx_v)

        gcp = [None] * _NBUF
        wcp = [None] * _NBUF

        def start_gather(c):
            b = c % _NBUF
            if wcp[b] is not None:
                wcp[b].wait()
                wcp[b] = None
            gcp[b] = pltpu.async_copy(
                table_hbm.at[idx_v.at[c]], bufs.at[b], gsem[b]
            )

        for c in range(min(_LA, n_chunks)):
            start_gather(c)
        for g in range(n_chunks):
            b = g % _NBUF
            gcp[b].wait()
            wcp[b] = pltpu.async_copy(
                bufs.at[b],
                out_hbm.at[bi, pl.ds(base + g * _C, _C)],
                wsem[b],
            )
            nxt = g + _LA
            if nxt < n_chunks:
                start_gather(nxt)
        for b in range(_NBUF):
            if wcp[b] is not None:
                wcp[b].wait()

    return gather_k


def kernel(position_ids, maps):
    B = position_ids.size
    V = maps.shape[0]
    tail = maps.shape[1:]
    ROW = 1
    for s in tail:
        ROW *= s
    table = maps.reshape(V, ROW // _LANE, _LANE)
    idx = position_ids.astype(jnp.int32).reshape(_NW, (B // _NW) // _C, _C)
    out = _build(position_ids.shape, V, ROW)(idx, table)
    return out.reshape(*position_ids.shape, *tail)
